# Initial kernel scaffold; baseline (speedup 1.0000x reference)
import functools

import jax
import jax.numpy as jnp
from jax import lax
from jax.experimental import pallas as pl
from jax.experimental.pallas import tpu as pltpu
from jax.experimental.pallas import tpu_sc as plsc

N = 10000
E = 320000
D_IN = 128
D_HID = 128
D_CLS = 40
D_CLS_PAD = 64

NC = 2
NS = 16
NW = NC * NS
L = 16

N_PAD = 10240
RPS = N_PAD // NS
C = 128
NCHUNK = E // C
BASE_TRIPS = NCHUNK // NW
EXTRA = NCHUNK % NW


def _sc_mesh():
  return plsc.VectorSubcoreMesh(
      core_axis_name="c", subcore_axis_name="s", num_cores=NC, num_subcores=NS
  )


@functools.partial(
    pl.kernel,
    out_type=jax.ShapeDtypeStruct((NC, N_PAD, L), jnp.float32),
    mesh=_sc_mesh(),
    scratch_types=[
        pltpu.VMEM((C,), jnp.int32),
        pltpu.VMEM((C, L), jnp.float32),
    ],
)
def _sc_degree(dst_hbm, out_hbm, didx_v, ones_v):
  cid = lax.axis_index("c")
  sid = lax.axis_index("s")
  wid = sid * NC + cid

  def _fill(val):
    def body(i, carry):
      ones_v[i, :] = jnp.full((L,), val, jnp.float32)
      return carry
    lax.fori_loop(0, C, body, 0)

  def _scoped(acc_sh):
    _fill(0.0)
    for j in range(RPS // C):
      pltpu.sync_copy(ones_v, acc_sh.at[pl.ds(sid * RPS + j * C, C)])
    _fill(1.0)
    plsc.subcore_barrier()

    def body(i, carry):
      c = wid + NW * i
      pltpu.sync_copy(dst_hbm.at[pl.ds(c * C, C)], didx_v)
      pltpu.sync_copy(ones_v, acc_sh.at[didx_v], add=True)
      return carry

    trips = BASE_TRIPS + jnp.where(wid < EXTRA, 1, 0)
    lax.fori_loop(0, trips, body, 0)
    plsc.subcore_barrier()
    pltpu.sync_copy(
        acc_sh.at[pl.ds(sid * RPS, RPS)], out_hbm.at[cid, pl.ds(sid * RPS, RPS)]
    )

  pl.run_scoped(_scoped, pltpu.VMEM_SHARED((N_PAD, L), jnp.float32))


def _make_sc_aggregate(D):
  @functools.partial(
      pl.kernel,
      out_type=jax.ShapeDtypeStruct((NC, N_PAD, D), jnp.float32),
      mesh=_sc_mesh(),
      scratch_types=[
          pltpu.VMEM((C,), jnp.int32),
          pltpu.VMEM((C,), jnp.int32),
          pltpu.VMEM((C, D), jnp.float32),
          pltpu.SemaphoreType.DMA,
      ],
  )
  def _sc_aggregate(table_hbm, src_hbm, dst_hbm, out_hbm, sidx_v, didx_v,
                    rows_v, sem):
    cid = lax.axis_index("c")
    sid = lax.axis_index("s")
    wid = sid * NC + cid

    def _scoped(acc_sh):
      def zbody(r, carry):
        for j in range(D // L):
          rows_v[r, pl.ds(j * L, L)] = jnp.zeros((L,), jnp.float32)
        return carry
      lax.fori_loop(0, C, zbody, 0)
      for j in range(RPS // C):
        pltpu.sync_copy(rows_v, acc_sh.at[pl.ds(sid * RPS + j * C, C)])
      plsc.subcore_barrier()

      def body(i, carry):
        c = wid + NW * i
        pltpu.sync_copy(src_hbm.at[pl.ds(c * C, C)], sidx_v)
        pltpu.sync_copy(dst_hbm.at[pl.ds(c * C, C)], didx_v)
        pltpu.async_copy(table_hbm.at[sidx_v], rows_v, sem).wait()
        pltpu.sync_copy(rows_v, acc_sh.at[didx_v], add=True)
        return carry

      trips = BASE_TRIPS + jnp.where(wid < EXTRA, 1, 0)
      lax.fori_loop(0, trips, body, 0)
      plsc.subcore_barrier()
      pltpu.sync_copy(
          acc_sh.at[pl.ds(sid * RPS, RPS)],
          out_hbm.at[cid, pl.ds(sid * RPS, RPS)],
      )

    pl.run_scoped(_scoped, pltpu.VMEM_SHARED((N_PAD, D), jnp.float32))

  return _sc_aggregate


_sc_aggregate_128 = _make_sc_aggregate(D_HID)
_sc_aggregate_64 = _make_sc_aggregate(D_CLS_PAD)


_R = 512
_GRID = N_PAD // _R


def _tc_layer0(feat, w0, d0, d1):
  def body(x_ref, w_ref, d0_ref, d1_ref, t_ref, n_ref):
    deg = d0_ref[...] + d1_ref[...]
    norm = lax.rsqrt(jnp.maximum(deg, 1.0))
    n_ref[...] = norm
    y = jnp.dot(x_ref[...], w_ref[...], preferred_element_type=jnp.float32)
    t_ref[...] = y * norm[:, 0:1]

  return pl.pallas_call(
      body,
      grid=(_GRID,),
      in_specs=[
          pl.BlockSpec((_R, D_IN), lambda i: (i, 0)),
          pl.BlockSpec((D_IN, D_HID), lambda i: (0, 0)),
          pl.BlockSpec((_R, L), lambda i: (i, 0)),
          pl.BlockSpec((_R, L), lambda i: (i, 0)),
      ],
      out_specs=[
          pl.BlockSpec((_R, D_HID), lambda i: (i, 0)),
          pl.BlockSpec((_R, L), lambda i: (i, 0)),
      ],
      out_shape=[
          jax.ShapeDtypeStruct((N_PAD, D_HID), jnp.float32),
          jax.ShapeDtypeStruct((N_PAD, L), jnp.float32),
      ],
  )(feat, w0, d0, d1)


def _tc_mid(p0, p1, norm, b, w, d_out):
  def body(p0_ref, p1_ref, n_ref, b_ref, w_ref, o_ref):
    nrm = n_ref[:, 0:1]
    h = (p0_ref[...] + p1_ref[...]) * nrm + b_ref[...]
    h = jnp.maximum(h, 0.0)
    o_ref[...] = (
        jnp.dot(h, w_ref[...], preferred_element_type=jnp.float32) * nrm
    )

  d_in = p0.shape[-1]
  return pl.pallas_call(
      body,
      grid=(_GRID,),
      in_specs=[
          pl.BlockSpec((_R, d_in), lambda i: (i, 0)),
          pl.BlockSpec((_R, d_in), lambda i: (i, 0)),
          pl.BlockSpec((_R, L), lambda i: (i, 0)),
          pl.BlockSpec((1, d_in), lambda i: (0, 0)),
          pl.BlockSpec((d_in, d_out), lambda i: (0, 0)),
      ],
      out_specs=pl.BlockSpec((_R, d_out), lambda i: (i, 0)),
      out_shape=jax.ShapeDtypeStruct((N_PAD, d_out), jnp.float32),
  )(p0, p1, norm, b, w)


def _tc_final(p0, p1, norm, b):
  rows = 400
  d = p0.shape[-1]

  def body(p0_ref, p1_ref, n_ref, b_ref, o_ref):
    o_ref[...] = (p0_ref[...] + p1_ref[...]) * n_ref[:, 0:1] + b_ref[...]

  return pl.pallas_call(
      body,
      grid=(N // rows,),
      in_specs=[
          pl.BlockSpec((rows, d), lambda i: (i, 0)),
          pl.BlockSpec((rows, d), lambda i: (i, 0)),
          pl.BlockSpec((rows, L), lambda i: (i, 0)),
          pl.BlockSpec((1, d), lambda i: (0, 0)),
      ],
      out_specs=pl.BlockSpec((rows, d), lambda i: (i, 0)),
      out_shape=jax.ShapeDtypeStruct((N, d), jnp.float32),
  )(p0, p1, norm, b)


def kernel(features, edge_index, W0, b0, W1, b1, W2, b2):
  src = edge_index[0]
  dst = edge_index[1]

  feat = jnp.pad(features, ((0, N_PAD - N), (0, 0)))
  w2p = jnp.pad(W2, ((0, 0), (0, D_CLS_PAD - D_CLS)))
  b2p = jnp.pad(b2, (0, D_CLS_PAD - D_CLS)).reshape(1, D_CLS_PAD)
  b0r = b0.reshape(1, D_HID)
  b1r = b1.reshape(1, D_HID)

  deg = _sc_degree(dst)
  t0, norm = _tc_layer0(feat, W0, deg[0], deg[1])

  a0 = _sc_aggregate_128(t0, src, dst)
  t1 = _tc_mid(a0[0], a0[1], norm, b0r, W1, D_HID)

  a1 = _sc_aggregate_128(t1, src, dst)
  t2 = _tc_mid(a1[0], a1[1], norm, b1r, w2p, D_CLS_PAD)

  a2 = _sc_aggregate_64(t2, src, dst)
  out = _tc_final(a2[0], a2[1], norm, b2p)
  return out[:, :D_CLS]


# speedup vs baseline: 5.2426x; 5.2426x over previous
import functools

import jax
import jax.numpy as jnp
from jax import lax
from jax.experimental import pallas as pl
from jax.experimental.pallas import tpu as pltpu
from jax.experimental.pallas import tpu_sc as plsc

N = 10000
E = 320000
D_IN = 128
D_HID = 128
D_CLS = 40
D_CLS_PAD = 128

NC = 2
NS = 16
NW = NC * NS
L = 16

N_PAD = 10240
RPS = N_PAD // NS
C = 128
NCHUNK = E // C
BASE_TRIPS = NCHUNK // NW
EXTRA = NCHUNK % NW


def _sc_mesh():
  return plsc.VectorSubcoreMesh(
      core_axis_name="c", subcore_axis_name="s", num_cores=NC, num_subcores=NS
  )


HR = N_PAD // 128


@functools.partial(
    pl.kernel,
    out_type=jax.ShapeDtypeStruct((NC, HR, 128), jnp.float32),
    mesh=_sc_mesh(),
    compiler_params=pltpu.CompilerParams(needs_layout_passes=False),
    scratch_types=[
        pltpu.VMEM((C,), jnp.int32),
        pltpu.VMEM((HR, 128), jnp.float32),
        pltpu.VMEM((HR,), jnp.int32),
        pltpu.VMEM_SHARED((HR, 128), jnp.float32),
    ],
)
def _sc_degree(dst_hbm, out_hbm, didx_v, hist_v, iota_v, acc_sh):
  cid = lax.axis_index("c")
  sid = lax.axis_index("s")
  wid = sid * NC + cid

  def zbody(r, carry):
    for j in range(8):
      hist_v[r, pl.ds(j * L, L)] = jnp.zeros((L,), jnp.float32)
    return carry
  lax.fori_loop(0, HR, zbody, 0)
  for j in range(HR // L):
    iota_v[pl.ds(j * L, L)] = lax.iota(jnp.int32, L) + j * L

  @pl.when(sid == 0)
  def _():
    pltpu.sync_copy(hist_v, acc_sh)
  plsc.subcore_barrier()

  ones = jnp.ones((L,), jnp.float32)

  def body(i, carry):
    c = wid + NW * i
    pltpu.sync_copy(dst_hbm.at[pl.ds(c * C, C)], didx_v)
    for k in range(C // L):
      idx = didx_v[pl.ds(k * L, L)]
      plsc.addupdate_scatter(hist_v, [idx >> 7, idx & 127], ones)
    return carry

  trips = BASE_TRIPS + jnp.where(wid < EXTRA, 1, 0)
  lax.fori_loop(0, trips, body, 0)

  pltpu.sync_copy(hist_v, acc_sh.at[iota_v], add=True)
  plsc.subcore_barrier()
  @pl.when(sid == 0)
  def _():
    pltpu.sync_copy(acc_sh, out_hbm.at[cid])


def _make_sc_aggregate(D):
  @functools.partial(
      pl.kernel,
      out_type=jax.ShapeDtypeStruct((NC, N_PAD, D), jnp.float32),
      mesh=_sc_mesh(),
      scratch_types=[
          pltpu.VMEM((C,), jnp.int32),
          pltpu.VMEM((C,), jnp.int32),
          pltpu.VMEM((C, D), jnp.float32),
          pltpu.VMEM_SHARED((N_PAD, D), jnp.float32),
          pltpu.SemaphoreType.DMA,
      ],
  )
  def _sc_aggregate(table_hbm, src_hbm, dst_hbm, out_hbm, sidx_v, didx_v,
                    rows_v, acc_sh, sem):
    cid = lax.axis_index("c")
    sid = lax.axis_index("s")
    wid = sid * NC + cid

    if True:
      def zbody(r, carry):
        for j in range(D // L):
          rows_v[r, pl.ds(j * L, L)] = jnp.zeros((L,), jnp.float32)
        return carry
      lax.fori_loop(0, C, zbody, 0)
      for j in range(RPS // C):
        pltpu.sync_copy(rows_v, acc_sh.at[pl.ds(sid * RPS + j * C, C)])
      plsc.subcore_barrier()

      def body(i, carry):
        c = wid + NW * i
        pltpu.sync_copy(src_hbm.at[pl.ds(c * C, C)], sidx_v)
        pltpu.sync_copy(dst_hbm.at[pl.ds(c * C, C)], didx_v)
        pltpu.async_copy(table_hbm.at[sidx_v], rows_v, sem).wait()
        pltpu.sync_copy(rows_v, acc_sh.at[didx_v], add=True)
        return carry

      trips = BASE_TRIPS + jnp.where(wid < EXTRA, 1, 0)
      lax.fori_loop(0, trips, body, 0)
      plsc.subcore_barrier()
      pltpu.sync_copy(
          acc_sh.at[pl.ds(sid * RPS, RPS)],
          out_hbm.at[cid, pl.ds(sid * RPS, RPS)],
      )

  return _sc_aggregate


_sc_aggregate_128 = _make_sc_aggregate(D_HID)


_R = 512
_GRID = N_PAD // _R


def _tc_layer0(feat, w0, d0, d1):
  def body(x_ref, w_ref, d0_ref, d1_ref, t_ref, n_ref):
    deg = d0_ref[...] + d1_ref[...]
    norm = lax.rsqrt(jnp.maximum(deg, 1.0))
    n_ref[...] = norm
    y = jnp.dot(x_ref[...], w_ref[...], preferred_element_type=jnp.float32)
    t_ref[...] = y * norm

  return pl.pallas_call(
      body,
      grid=(_GRID,),
      in_specs=[
          pl.BlockSpec((_R, D_IN), lambda i: (i, 0)),
          pl.BlockSpec((D_IN, D_HID), lambda i: (0, 0)),
          pl.BlockSpec((_R, 1), lambda i: (i, 0)),
          pl.BlockSpec((_R, 1), lambda i: (i, 0)),
      ],
      out_specs=[
          pl.BlockSpec((_R, D_HID), lambda i: (i, 0)),
          pl.BlockSpec((_R, 1), lambda i: (i, 0)),
      ],
      out_shape=[
          jax.ShapeDtypeStruct((N_PAD, D_HID), jnp.float32),
          jax.ShapeDtypeStruct((N_PAD, 1), jnp.float32),
      ],
  )(feat, w0, d0, d1)


def _tc_mid(p0, p1, norm, b, w, d_out):
  def body(p0_ref, p1_ref, n_ref, b_ref, w_ref, o_ref):
    nrm = n_ref[...]
    h = (p0_ref[...] + p1_ref[...]) * nrm + b_ref[...]
    h = jnp.maximum(h, 0.0)
    o_ref[...] = (
        jnp.dot(h, w_ref[...], preferred_element_type=jnp.float32) * nrm
    )

  d_in = p0.shape[-1]
  return pl.pallas_call(
      body,
      grid=(_GRID,),
      in_specs=[
          pl.BlockSpec((_R, d_in), lambda i: (i, 0)),
          pl.BlockSpec((_R, d_in), lambda i: (i, 0)),
          pl.BlockSpec((_R, 1), lambda i: (i, 0)),
          pl.BlockSpec((1, d_in), lambda i: (0, 0)),
          pl.BlockSpec((d_in, d_out), lambda i: (0, 0)),
      ],
      out_specs=pl.BlockSpec((_R, d_out), lambda i: (i, 0)),
      out_shape=jax.ShapeDtypeStruct((N_PAD, d_out), jnp.float32),
  )(p0, p1, norm, b, w)


def _tc_final(p0, p1, norm, b):
  rows = 400
  d = p0.shape[-1]

  def body(p0_ref, p1_ref, n_ref, b_ref, o_ref):
    o_ref[...] = (p0_ref[...] + p1_ref[...]) * n_ref[...] + b_ref[...]

  return pl.pallas_call(
      body,
      grid=(N // rows,),
      in_specs=[
          pl.BlockSpec((rows, d), lambda i: (i, 0)),
          pl.BlockSpec((rows, d), lambda i: (i, 0)),
          pl.BlockSpec((rows, 1), lambda i: (i, 0)),
          pl.BlockSpec((1, d), lambda i: (0, 0)),
      ],
      out_specs=pl.BlockSpec((rows, d), lambda i: (i, 0)),
      out_shape=jax.ShapeDtypeStruct((N, d), jnp.float32),
  )(p0, p1, norm, b)


def kernel(features, edge_index, W0, b0, W1, b1, W2, b2):
  src = edge_index[0]
  dst = edge_index[1]

  feat = jnp.pad(features, ((0, N_PAD - N), (0, 0)))
  w2p = jnp.pad(W2, ((0, 0), (0, D_CLS_PAD - D_CLS)))
  b2p = jnp.pad(b2, (0, D_CLS_PAD - D_CLS)).reshape(1, D_CLS_PAD)
  b0r = b0.reshape(1, D_HID)
  b1r = b1.reshape(1, D_HID)

  deg = _sc_degree(dst).reshape(NC, N_PAD, 1)
  t0, norm = _tc_layer0(feat, W0, deg[0], deg[1])

  a0 = _sc_aggregate_128(t0, src, dst)
  t1 = _tc_mid(a0[0], a0[1], norm, b0r, W1, D_HID)

  a1 = _sc_aggregate_128(t1, src, dst)
  t2 = _tc_mid(a1[0], a1[1], norm, b1r, w2p, D_CLS_PAD)

  a2 = _sc_aggregate_128(t2, src, dst)
  out = _tc_final(a2[0], a2[1], norm, b2p)
  return out[:, :D_CLS]
